# baseline (device time: 26474 ns/iter reference)
import jax
import jax.numpy as jnp
from jax import lax
from jax.experimental import pallas as pl
from jax.experimental.pallas import tpu as pltpu

P = 32


def kernel(x):
    m, n = x.shape
    c = m // P

    def body(x_ref, out_ref, stage_ref, rs_ref, red_ref, ag_ref,
             rs_send_sems, rs_recv_sems, ag_send_sems, ag_recv_sems):
        my = lax.axis_index("i")

        barrier_sem = pltpu.get_barrier_semaphore()
        for o in range(1, P):
            peer = lax.rem(my + o, P)
            pl.semaphore_signal(
                barrier_sem, inc=1,
                device_id=(peer,), device_id_type=pl.DeviceIdType.MESH,
            )
        pl.semaphore_wait(barrier_sem, P - 1)

        for o in range(1, P):
            e = lax.rem(my + o, P)
            stage_ref[o - 1, :, :] = (
                x_ref[pl.ds(e * c, c), :].astype(jnp.bfloat16)
            )

        rs_sends = []
        for o in range(1, P):
            e = lax.rem(my + o, P)
            rdma = pltpu.make_async_remote_copy(
                src_ref=stage_ref.at[o - 1],
                dst_ref=rs_ref.at[o - 1],
                send_sem=rs_send_sems.at[o - 1],
                recv_sem=rs_recv_sems.at[o - 1],
                device_id=(e,),
                device_id_type=pl.DeviceIdType.MESH,
            )
            rdma.start()
            rs_sends.append(rdma)

        red = x_ref[pl.ds(my * c, c), :]
        for o in range(1, P):
            rs_sends[o - 1].wait_recv()
            red = red + rs_ref[o - 1, :, :].astype(jnp.float32)
        red_ref[...] = red.astype(jnp.bfloat16)
        out_ref[pl.ds(my * c, c), :] = red_ref[...]

        ag_sends = []
        for o in range(1, P):
            e = lax.rem(my + o, P)
            rdma = pltpu.make_async_remote_copy(
                src_ref=red_ref,
                dst_ref=ag_ref.at[o - 1],
                send_sem=ag_send_sems.at[o - 1],
                recv_sem=ag_recv_sems.at[o - 1],
                device_id=(e,),
                device_id_type=pl.DeviceIdType.MESH,
            )
            rdma.start()
            ag_sends.append(rdma)

        for r in rs_sends:
            r.wait_send()

        for o in range(1, P):
            ag_sends[o - 1].wait_recv()
            s = lax.rem(my - o + P, P)
            out_ref[pl.ds(s * c, c), :] = ag_ref[o - 1, :, :]

        for r in ag_sends:
            r.wait_send()

    return pl.pallas_call(
        body,
        out_shape=jax.ShapeDtypeStruct((m, n), jnp.bfloat16),
        in_specs=[pl.BlockSpec(memory_space=pltpu.VMEM)],
        out_specs=pl.BlockSpec(memory_space=pltpu.VMEM),
        scratch_shapes=[
            pltpu.VMEM((P - 1, c, n), jnp.bfloat16),
            pltpu.VMEM((P - 1, c, n), jnp.bfloat16),
            pltpu.VMEM((c, n), jnp.bfloat16),
            pltpu.VMEM((P - 1, c, n), jnp.bfloat16),
            pltpu.SemaphoreType.DMA((P - 1,)),
            pltpu.SemaphoreType.DMA((P - 1,)),
            pltpu.SemaphoreType.DMA((P - 1,)),
            pltpu.SemaphoreType.DMA((P - 1,)),
        ],
        compiler_params=pltpu.CompilerParams(collective_id=0),
    )(x)


# device time: 26353 ns/iter; 1.0046x vs baseline; 1.0046x over previous
import jax
import jax.numpy as jnp
from jax import lax
from jax.experimental import pallas as pl
from jax.experimental.pallas import tpu as pltpu

P = 32


def kernel(x):
    m, n = x.shape
    c = m // P

    def body(x_ref, out_ref, stage_ref, rs_ref, red_ref,
             rs_send_sems, rs_recv_sems, ag_send_sems, ag_recv_sems):
        my = lax.axis_index("i")

        barrier_sem = pltpu.get_barrier_semaphore()
        for o in range(1, P):
            peer = lax.rem(my + o, P)
            pl.semaphore_signal(
                barrier_sem, inc=1,
                device_id=(peer,), device_id_type=pl.DeviceIdType.MESH,
            )

        for o in range(1, P):
            e = lax.rem(my + o, P)
            stage_ref[o - 1, :, :] = (
                x_ref[pl.ds(e * c, c), :].astype(jnp.bfloat16)
            )

        pl.semaphore_wait(barrier_sem, P - 1)

        rs_sends = []
        for o in range(1, P):
            e = lax.rem(my + o, P)
            rdma = pltpu.make_async_remote_copy(
                src_ref=stage_ref.at[o - 1],
                dst_ref=rs_ref.at[o - 1],
                send_sem=rs_send_sems.at[o - 1],
                recv_sem=rs_recv_sems.at[o - 1],
                device_id=(e,),
                device_id_type=pl.DeviceIdType.MESH,
            )
            rdma.start()
            rs_sends.append(rdma)

        red = x_ref[pl.ds(my * c, c), :]
        for o in range(1, P):
            rs_sends[o - 1].wait_recv()
            red = red + rs_ref[o - 1, :, :].astype(jnp.float32)
        red_ref[...] = red.astype(jnp.bfloat16)
        out_ref[pl.ds(my * c, c), :] = red_ref[...]

        ag_sends = []
        for o in range(1, P):
            e = lax.rem(my + o, P)
            rdma = pltpu.make_async_remote_copy(
                src_ref=red_ref,
                dst_ref=out_ref.at[pl.ds(my * c, c), :],
                send_sem=ag_send_sems.at[o - 1],
                recv_sem=ag_recv_sems.at[o - 1],
                device_id=(e,),
                device_id_type=pl.DeviceIdType.MESH,
            )
            rdma.start()
            ag_sends.append(rdma)

        for r in rs_sends:
            r.wait_send()

        for r in ag_sends:
            r.wait_recv()
        for r in ag_sends:
            r.wait_send()

    return pl.pallas_call(
        body,
        out_shape=jax.ShapeDtypeStruct((m, n), jnp.bfloat16),
        in_specs=[pl.BlockSpec(memory_space=pltpu.VMEM)],
        out_specs=pl.BlockSpec(memory_space=pltpu.VMEM),
        scratch_shapes=[
            pltpu.VMEM((P - 1, c, n), jnp.bfloat16),
            pltpu.VMEM((P - 1, c, n), jnp.bfloat16),
            pltpu.VMEM((c, n), jnp.bfloat16),
            pltpu.SemaphoreType.DMA((P - 1,)),
            pltpu.SemaphoreType.DMA((P - 1,)),
            pltpu.SemaphoreType.DMA((P - 1,)),
            pltpu.SemaphoreType.DMA((P - 1,)),
        ],
        compiler_params=pltpu.CompilerParams(collective_id=0),
    )(x)
